# Initial kernel scaffold; baseline (speedup 1.0000x reference)
import jax
import jax.numpy as jnp
from jax.experimental import pallas as pl


def kernel(output, target):
    raise NotImplementedError("write your pallas kernel here")


# speedup vs baseline: 3.9480x; 3.9480x over previous
import jax
import jax.numpy as jnp
from jax import lax
from jax.experimental import pallas as pl
from jax.experimental.pallas import tpu as pltpu

_B = 16384
_R = 128
_C = 128

_CONST_CACHE = {}


def _rd_constants():
    if "rd" not in _CONST_CACHE:
        rd = jax.random.normal(jax.random.key(42), (_B + 1,), jnp.float32)
        rd = jnp.maximum(jnp.sort(-jnp.exp(rd)), -1.0)
        rd0 = rd[:_B].reshape(_R, _C)
        rd1 = rd[1:].reshape(_R, _C)
        _CONST_CACHE["rd"] = (rd0, rd1)
    return _CONST_CACHE["rd"]


def _roll(x, shift, axis):
    return jnp.roll(x, shift, axis)


def _ce_stage(x, bs, d_lane, lane_iota, glob_i):
    is_hi = (lane_iota & d_lane) != 0
    asc = (glob_i & bs) == 0
    partner = jnp.where(is_hi, _roll(x, d_lane, 1), _roll(x, -d_lane, 1))
    keep_min = asc != is_hi
    return jnp.where(keep_min, jnp.minimum(x, partner), jnp.maximum(x, partner))


def _bitonic_sort(v):
    x = v
    row_n = lax.broadcasted_iota(jnp.int32, (_R, _C), 0)
    col_n = lax.broadcasted_iota(jnp.int32, (_R, _C), 1)
    i_n = row_n * _C + col_n
    i_t = col_n * _C + row_n
    for k in range(1, 15):
        bs = 1 << k
        js = list(range(k - 1, -1, -1))
        row_js = [j for j in js if j >= 7]
        lane_js = [j for j in js if j < 7]
        if row_js:
            x = x.T
            for j in row_js:
                x = _ce_stage(x, bs, 1 << (j - 7), col_n, i_t)
            x = x.T
        for j in lane_js:
            x = _ce_stage(x, bs, 1 << j, col_n, i_n)
    return x


def _body(out_ref, tgt_ref, rd0_ref, rd1_ref, loss_ref):
    outp = out_ref[...]
    tgt = tgt_ref[...].astype(jnp.float32)
    s = outp * (2.0 * tgt - 1.0)
    v = jnp.minimum(s, 1.0)
    s_sum = jnp.sum(s)
    r0 = jnp.sum((s < 0.0).astype(jnp.int32))
    w = _bitonic_sort(v)
    row = lax.broadcasted_iota(jnp.int32, (_R, _C), 0)
    col = lax.broadcasted_iota(jnp.int32, (_R, _C), 1)
    k = row * _C + col
    sel = jnp.where(k < r0, rd0_ref[...], rd1_ref[...])
    loss = (jnp.sum(sel * w) - 1e-6 * s_sum) / _B
    loss_ref[0, 0] = loss


def _pallas_loss(outp, tgt, rd0, rd1, interpret=False):
    return pl.pallas_call(
        _body,
        out_shape=jax.ShapeDtypeStruct((1, 1), jnp.float32),
        out_specs=pl.BlockSpec(memory_space=pltpu.SMEM),
        interpret=interpret,
    )(outp, tgt, rd0, rd1)


def kernel(output, target, interpret=False):
    rd0, rd1 = _rd_constants()
    outp = output.reshape(_R, _C).astype(jnp.float32)
    tgt = target.reshape(_R, _C).astype(jnp.int32)
    res = _pallas_loss(outp, tgt, rd0, rd1, interpret=interpret)
    return res[0, 0]


# speedup vs baseline: 4.4213x; 1.1199x over previous
import jax
import jax.numpy as jnp
from jax import lax
from jax.experimental import pallas as pl
from jax.experimental.pallas import tpu as pltpu

_B = 16384
_R = 128
_C = 128

_CONST_CACHE = {}


def _rd_constants():
    if "rd" not in _CONST_CACHE:
        rd = jax.random.normal(jax.random.key(42), (_B + 1,), jnp.float32)
        rd = jnp.maximum(jnp.sort(-jnp.exp(rd)), -1.0)
        rd0 = rd[:_B].reshape(_R, _C)
        rd1 = rd[1:].reshape(_R, _C)
        _CONST_CACHE["rd"] = (rd0, rd1)
    return _CONST_CACHE["rd"]


def _roll(x, shift, axis):
    return jnp.roll(x, shift, axis)


def _roll_stage(xs, bs, d, axis, dist, col8, sub8):
    it = col8 if axis == 1 else sub8
    is_hi = (it & dist) != 0
    is_lo = jnp.logical_not(is_hi)
    out = []
    if bs <= 512:
        if bs <= 64:
            asc = (col8 & bs) == 0
        else:
            asc = (sub8 & (bs // 128)) == 0
        km = asc == is_lo
        for xi in xs:
            p = jnp.where(is_hi, _roll(xi, dist, axis), _roll(xi, -dist, axis))
            out.append(jnp.where(km, jnp.minimum(xi, p), jnp.maximum(xi, p)))
    else:
        for i, xi in enumerate(xs):
            rm = _roll(xi, -dist, axis)
            rp = _roll(xi, dist, axis)
            if (i & (bs // 1024)) != 0:
                out.append(jnp.where(is_lo, jnp.maximum(xi, rm),
                                     jnp.minimum(xi, rp)))
            else:
                out.append(jnp.where(is_lo, jnp.minimum(xi, rm),
                                     jnp.maximum(xi, rp)))
    return out


def _vreg_stage(xs, bs, d):
    dv = d // 1024
    out = list(xs)
    for i in range(16):
        if (i & dv) == 0:
            j = i + dv
            mn = jnp.minimum(xs[i], xs[j])
            mx = jnp.maximum(xs[i], xs[j])
            if (i & (bs // 1024)) != 0:
                out[i], out[j] = mx, mn
            else:
                out[i], out[j] = mn, mx
    return out


def _bitonic_sort(v):
    xs = [v[8 * i:8 * (i + 1), :] for i in range(16)]
    sub8 = lax.broadcasted_iota(jnp.int32, (8, _C), 0)
    col8 = lax.broadcasted_iota(jnp.int32, (8, _C), 1)
    for k in range(1, 15):
        bs = 1 << k
        for j in range(k - 1, -1, -1):
            d = 1 << j
            if d <= 64:
                xs = _roll_stage(xs, bs, d, 1, d, col8, sub8)
            elif d <= 512:
                xs = _roll_stage(xs, bs, d, 0, d // 128, col8, sub8)
            else:
                xs = _vreg_stage(xs, bs, d)
    return jnp.concatenate(xs, axis=0)


def _body(out_ref, tgt_ref, rd0_ref, rd1_ref, loss_ref):
    outp = out_ref[...]
    tgt = tgt_ref[...].astype(jnp.float32)
    s = outp * (2.0 * tgt - 1.0)
    v = jnp.minimum(s, 1.0)
    s_sum = jnp.sum(s)
    r0 = jnp.sum((s < 0.0).astype(jnp.int32))
    w = _bitonic_sort(v)
    row = lax.broadcasted_iota(jnp.int32, (_R, _C), 0)
    col = lax.broadcasted_iota(jnp.int32, (_R, _C), 1)
    k = row * _C + col
    sel = jnp.where(k < r0, rd0_ref[...], rd1_ref[...])
    loss = (jnp.sum(sel * w) - 1e-6 * s_sum) / _B
    loss_ref[0, 0] = loss


def _pallas_loss(outp, tgt, rd0, rd1, interpret=False):
    return pl.pallas_call(
        _body,
        out_shape=jax.ShapeDtypeStruct((1, 1), jnp.float32),
        out_specs=pl.BlockSpec(memory_space=pltpu.SMEM),
        interpret=interpret,
    )(outp, tgt, rd0, rd1)


def kernel(output, target, interpret=False):
    rd0, rd1 = _rd_constants()
    outp = output.reshape(_R, _C).astype(jnp.float32)
    tgt = target.reshape(_R, _C).astype(jnp.int32)
    res = _pallas_loss(outp, tgt, rd0, rd1, interpret=interpret)
    return res[0, 0]
